# baseline (speedup 1.0000x reference)
import functools

import numpy as np
import jax
import jax.numpy as jnp
from jax import lax
from jax.experimental import pallas as pl
from jax.experimental.pallas import tpu as pltpu
from jax.experimental.pallas import tpu_sc as plsc

_VOCAB = 100000
_EMBED = 128
_WINDOW = 200
_BATCH = 1024
_SCALE = float(np.sqrt(float(_EMBED)))

_NC = 2
_NS = 16
_NW = _NC * _NS
_ROWS_PER_W = _BATCH // _NW
_HALF = _WINDOW // 2
_PAIRS = _ROWS_PER_W // 2


def _positional_encoding(length, depth):
    pos = np.arange(length)[:, np.newaxis]
    i = np.arange(depth)[np.newaxis, :]
    val = pos / 10000 ** (2 * (i // 2) / depth)
    pe = np.concatenate([np.sin(val[:, 0::2]), np.cos(val[:, 1::2])], axis=-1)
    return pe.astype(np.float32)


_POS = _positional_encoding(_WINDOW, _EMBED)


def _sc_body(x_hbm, pos_hbm, table_hbm, out_hbm,
             idx_v, rows0, rows1, pos_v, sg0, sg1, sw0, sw1):
    wid = lax.axis_index("s") * _NC + lax.axis_index("c")
    base = wid * _ROWS_PER_W
    pltpu.sync_copy(pos_hbm, pos_v)
    pltpu.sync_copy(x_hbm.at[wid], idx_v)

    def start_gather(r, buf, sem):
        pltpu.async_copy(table_hbm.at[idx_v.at[r, 0]],
                         buf.at[pl.ds(0, _HALF)], sem)
        pltpu.async_copy(table_hbm.at[idx_v.at[r, 1]],
                         buf.at[pl.ds(_HALF, _HALF)], sem)

    def wait_gather(buf, sem):
        pltpu.make_async_copy(table_hbm.at[pl.ds(0, _WINDOW)], buf, sem).wait()

    def compute(buf):
        def tok(t, c):
            for u in range(2):
                tt = t * 2 + u
                for v in range(_EMBED // 16):
                    sl = (tt, pl.ds(v * 16, 16))
                    buf[sl] = buf[sl] * _SCALE + pos_v[sl]
            return c
        lax.fori_loop(0, _WINDOW // 2, tok, 0)

    start_gather(0, rows0, sg0)

    def pair(j, carry):
        start_gather(2 * j + 1, rows1, sg1)
        wait_gather(rows0, sg0)

        @pl.when(j < _PAIRS - 1)
        def _():
            start_gather(2 * j + 2, rows0, sg0)
        wait_gather(rows1, sg1)
        return carry

    lax.fori_loop(0, _PAIRS, pair, 0)
    pltpu.async_copy(pos_v, out_hbm.at[base], sw0)
    pltpu.make_async_copy(pos_v, out_hbm.at[0], sw0).wait()


@jax.jit
def kernel(x, table):
    x4 = x.reshape(_NW, _ROWS_PER_W, 2, _HALF)
    pos = jnp.asarray(_POS)
    table = lax.bitcast_convert_type(
        table.astype(jnp.bfloat16).reshape(_VOCAB, _EMBED // 2, 2), jnp.int32)
    mesh = plsc.VectorSubcoreMesh(core_axis_name="c", subcore_axis_name="s")
    call = functools.partial(
        pl.kernel,
        mesh=mesh,
        compiler_params=pltpu.CompilerParams(use_tc_tiling_on_sc=False),
        out_type=jax.ShapeDtypeStruct((_BATCH, _WINDOW, _EMBED), jnp.float32),
        scratch_types=[
            pltpu.VMEM((_ROWS_PER_W, 2, _HALF), jnp.int32),
            pltpu.VMEM((_WINDOW, _EMBED // 2), jnp.int32),
            pltpu.VMEM((_WINDOW, _EMBED // 2), jnp.int32),
            pltpu.VMEM((_WINDOW, _EMBED), jnp.float32),
            pltpu.SemaphoreType.DMA,
            pltpu.SemaphoreType.DMA,
            pltpu.SemaphoreType.DMA,
            pltpu.SemaphoreType.DMA,
        ],
    )(_sc_body)
    return call(x4, pos, table)


# speedup vs baseline: 2.4557x; 2.4557x over previous
import functools

import numpy as np
import jax
import jax.numpy as jnp
from jax import lax
from jax.experimental import pallas as pl
from jax.experimental.pallas import tpu as pltpu
from jax.experimental.pallas import tpu_sc as plsc

_VOCAB = 100000
_EMBED = 128
_WINDOW = 200
_BATCH = 1024
_SCALE = float(np.sqrt(float(_EMBED)))

_NC = 2
_NS = 16
_NW = _NC * _NS
_HALF = _WINDOW // 2
_BLOCKS_PER_W = _BATCH * 2 // _NW
_NSLOT = 4
_GROUPS = _BLOCKS_PER_W // _NSLOT


def _positional_encoding(length, depth):
    pos = np.arange(length)[:, np.newaxis]
    i = np.arange(depth)[np.newaxis, :]
    val = pos / 10000 ** (2 * (i // 2) / depth)
    pe = np.concatenate([np.sin(val[:, 0::2]), np.cos(val[:, 1::2])], axis=-1)
    return pe.astype(np.float32)


_POS = _positional_encoding(_WINDOW, _EMBED)


def _sc_body(x_hbm, pos_hbm, table_hbm, out_hbm, idx_v, *rest):
    bufs = rest[:_NSLOT]
    pos_v = rest[_NSLOT]
    sgs = rest[_NSLOT + 1:2 * _NSLOT + 1]
    sws = rest[2 * _NSLOT + 1:]
    wid = lax.axis_index("s") * _NC + lax.axis_index("c")
    base = wid * _BLOCKS_PER_W
    pltpu.sync_copy(pos_hbm, pos_v)
    pltpu.sync_copy(x_hbm.at[wid], idx_v)

    def start_gather(b, s):
        pltpu.async_copy(table_hbm.at[idx_v.at[b]], bufs[s], sgs[s])

    def wait_gather(s):
        pltpu.make_async_copy(out_hbm.at[0], bufs[s], sgs[s]).wait()

    def wb(b, s):
        pltpu.async_copy(bufs[s], out_hbm.at[base + b], sws[s])

    def wait_wb(s):
        pltpu.make_async_copy(bufs[s], out_hbm.at[0], sws[s]).wait()

    def compute(s, parity):
        buf = bufs[s]
        pbase = parity * _HALF

        def tok(t, c):
            for u in range(2):
                tt = t * 2 + u
                for v in range(_EMBED // 16):
                    buf[tt, pl.ds(v * 16, 16)] = (
                        buf[tt, pl.ds(v * 16, 16)] * _SCALE
                        + pos_v[pbase + tt, pl.ds(v * 16, 16)])
            return c
        lax.fori_loop(0, _HALF // 2, tok, 0)

    for s in range(_NSLOT):
        start_gather(s, s)

    def grp(j, carry):
        for s in range(_NSLOT):
            b = _NSLOT * j + s
            wait_gather(s)
            compute(s, lax.rem(b, 2))
            wb(b, s)

            @pl.when(j < _GROUPS - 1)
            def _():
                wait_wb(s)
                start_gather(b + _NSLOT, s)
        return carry

    lax.fori_loop(0, _GROUPS, grp, 0)
    for s in range(_NSLOT):
        wait_wb(s)


@jax.jit
def kernel(x, table):
    x3 = x.reshape(_NW, _BLOCKS_PER_W, _HALF)
    pos = jnp.asarray(_POS)
    mesh = plsc.VectorSubcoreMesh(core_axis_name="c", subcore_axis_name="s")
    call = functools.partial(
        pl.kernel,
        mesh=mesh,
        out_type=jax.ShapeDtypeStruct((_BATCH * 2, _HALF, _EMBED),
                                      jnp.float32),
        scratch_types=(
            [pltpu.VMEM((_BLOCKS_PER_W, _HALF), jnp.int32)]
            + [pltpu.VMEM((_HALF, _EMBED), jnp.float32)] * _NSLOT
            + [pltpu.VMEM((_WINDOW, _EMBED), jnp.float32)]
            + [pltpu.SemaphoreType.DMA] * (2 * _NSLOT)
        ),
    )(_sc_body)
    out = call(x3, pos, table)
    return out.reshape(_BATCH, _WINDOW, _EMBED)


# speedup vs baseline: 4.5073x; 1.8354x over previous
import functools

import numpy as np
import jax
import jax.numpy as jnp
from jax import lax
from jax.experimental import pallas as pl
from jax.experimental.pallas import tpu as pltpu
from jax.experimental.pallas import tpu_sc as plsc

_VOCAB = 100000
_EMBED = 128
_WINDOW = 200
_BATCH = 1024
_SCALE = float(np.sqrt(float(_EMBED)))

_NC = 2
_NS = 16
_NW = _NC * _NS
_ROWS_PER_W = _BATCH // _NW
_HALF = _WINDOW // 2
_PAIRS = _ROWS_PER_W // 2


def _positional_encoding(length, depth):
    pos = np.arange(length)[:, np.newaxis]
    i = np.arange(depth)[np.newaxis, :]
    val = pos / 10000 ** (2 * (i // 2) / depth)
    pe = np.concatenate([np.sin(val[:, 0::2]), np.cos(val[:, 1::2])], axis=-1)
    return pe.astype(np.float32)


_POS = _positional_encoding(_WINDOW, _EMBED)


def _sc_body(x_hbm, pos_hbm, table_hbm, out_hbm,
             idx_v, rows0, rows1, pos_v, sg0, sg1, sw0, sw1):
    wid = lax.axis_index("s") * _NC + lax.axis_index("c")
    base = wid * _ROWS_PER_W
    pltpu.sync_copy(pos_hbm, pos_v)
    pltpu.sync_copy(x_hbm.at[wid], idx_v)

    def start_gather(r, buf, sem):
        pltpu.async_copy(table_hbm.at[idx_v.at[r, 0]],
                         buf.at[pl.ds(0, _HALF)], sem)
        pltpu.async_copy(table_hbm.at[idx_v.at[r, 1]],
                         buf.at[pl.ds(_HALF, _HALF)], sem)

    def wait_gather(buf, sem):
        pltpu.make_async_copy(table_hbm.at[pl.ds(0, _WINDOW)], buf, sem).wait()

    def start_wb(buf, r, sem):
        pltpu.async_copy(buf, out_hbm.at[base + r], sem)

    def wait_wb(buf, sem):
        pltpu.make_async_copy(buf, out_hbm.at[0], sem).wait()

    def compute(buf):
        def tok(t, c):
            for u in range(2):
                tt = t * 2 + u
                for v in range(_EMBED // 16):
                    sl = (tt, pl.ds(v * 16, 16))
                    buf[sl] = buf[sl] * _SCALE + pos_v[sl]
            return c
        lax.fori_loop(0, _WINDOW // 2, tok, 0)

    start_gather(0, rows0, sg0)

    def pair(j, carry):
        @pl.when(j > 0)
        def _():
            wait_wb(rows1, sw1)
        start_gather(2 * j + 1, rows1, sg1)
        wait_gather(rows0, sg0)
        compute(rows0)
        start_wb(rows0, 2 * j, sw0)

        @pl.when(j < _PAIRS - 1)
        def _():
            wait_wb(rows0, sw0)
            start_gather(2 * j + 2, rows0, sg0)
        wait_gather(rows1, sg1)
        compute(rows1)
        start_wb(rows1, 2 * j + 1, sw1)
        return carry

    lax.fori_loop(0, _PAIRS, pair, 0)
    wait_wb(rows0, sw0)
    wait_wb(rows1, sw1)


@jax.jit
def kernel(x, table):
    x4 = x.reshape(_NW, _ROWS_PER_W, 2, _HALF)
    pos = jnp.asarray(_POS)
    mesh = plsc.VectorSubcoreMesh(core_axis_name="c", subcore_axis_name="s")
    call = functools.partial(
        pl.kernel,
        mesh=mesh,
        out_type=jax.ShapeDtypeStruct((_BATCH, _WINDOW, _EMBED), jnp.float32),
        scratch_types=[
            pltpu.VMEM((_ROWS_PER_W, 2, _HALF), jnp.int32),
            pltpu.VMEM((_WINDOW, _EMBED), jnp.float32),
            pltpu.VMEM((_WINDOW, _EMBED), jnp.float32),
            pltpu.VMEM((_WINDOW, _EMBED), jnp.float32),
            pltpu.SemaphoreType.DMA,
            pltpu.SemaphoreType.DMA,
            pltpu.SemaphoreType.DMA,
            pltpu.SemaphoreType.DMA,
        ],
    )(_sc_body)
    return call(x4, pos, table)
